# baseline (speedup 1.0000x reference)
import functools
import jax
import jax.numpy as jnp
from jax import lax
from jax.experimental import pallas as pl
from jax.experimental.pallas import tpu as pltpu
from jax.experimental.pallas import tpu_sc as plsc

_NUM_EXPERTS = 16
_COEF = 0.01
_EPS = 1e-10
_NW = 32
_L = 16


def _matmul_body(x_ref, w_ref, out_ref):
    out_ref[...] = jnp.dot(x_ref[...], w_ref[...],
                           preferred_element_type=jnp.float32)


def _tc_logits(x, w_gate):
    n_tokens, d_model = x.shape
    tile = 2048
    return pl.pallas_call(
        _matmul_body,
        grid=(n_tokens // tile,),
        in_specs=[
            pl.BlockSpec((tile, d_model), lambda i: (i, 0)),
            pl.BlockSpec((d_model, _NUM_EXPERTS), lambda i: (0, 0)),
        ],
        out_specs=pl.BlockSpec((tile, _NUM_EXPERTS), lambda i: (i, 0)),
        out_shape=jax.ShapeDtypeStruct((n_tokens, _NUM_EXPERTS), jnp.float32),
    )(x, w_gate)


def _make_sc_route(n_tokens):
    tpw = n_tokens // _NW
    n_groups = tpw // _L
    mesh = plsc.VectorSubcoreMesh(core_axis_name="c", subcore_axis_name="s")

    @functools.partial(
        pl.kernel, mesh=mesh,
        out_type=[
            jax.ShapeDtypeStruct((n_tokens, _NUM_EXPERTS), jnp.float32),
            jax.ShapeDtypeStruct((2 * _NW, _NUM_EXPERTS), jnp.float32),
        ],
        scratch_types=[
            pltpu.VMEM((tpw, _NUM_EXPERTS), jnp.float32),
            pltpu.VMEM((tpw, _NUM_EXPERTS), jnp.float32),
            pltpu.VMEM((2, _NUM_EXPERTS), jnp.float32),
        ],
        compiler_params=pltpu.CompilerParams(needs_layout_passes=False),
    )
    def route(logits_hbm, gates_hbm, parts_hbm, log_v, out_v, part_v):
        wid = lax.axis_index("s") * 2 + lax.axis_index("c")
        base = wid * tpw
        pltpu.sync_copy(logits_hbm.at[pl.ds(base, tpw), :], log_v)

        lane = lax.iota(jnp.int32, _L)
        zeros = jnp.zeros((_L,), jnp.float32)
        ones = jnp.ones((_L,), jnp.float32)
        neg_inf = jnp.full((_L,), -jnp.inf, jnp.float32)

        part_v[0, :] = zeros
        part_v[1, :] = zeros

        def group_body(g, _):
            row = g * _L + lane
            m1 = plsc.load_gather(log_v, [row, jnp.zeros((_L,), jnp.int32)])
            e1 = jnp.zeros((_L,), jnp.int32)
            m2 = neg_inf
            e2 = jnp.zeros((_L,), jnp.int32)
            for e in range(1, _NUM_EXPERTS):
                ev = jnp.full((_L,), e, jnp.int32)
                lv = plsc.load_gather(log_v, [row, ev])
                new1 = lv > m1
                new2 = jnp.logical_and(jnp.logical_not(new1), lv > m2)
                m2 = jnp.where(new1, m1, jnp.where(new2, lv, m2))
                e2 = jnp.where(new1, e1, jnp.where(new2, ev, e2))
                m1 = jnp.where(new1, lv, m1)
                e1 = jnp.where(new1, ev, e1)

            d = jnp.exp(m2 - m1)
            s = ones + d
            g1 = ones / s
            g2 = d / s

            for j in range(_L):
                out_v[g * _L + j, :] = zeros
            plsc.addupdate_scatter(out_v, [row, e1], g1)
            plsc.addupdate_scatter(out_v, [row, e2], g2)

            plsc.addupdate_scatter(part_v, [jnp.zeros((_L,), jnp.int32), e1],
                                   g1)
            plsc.addupdate_scatter(part_v, [jnp.zeros((_L,), jnp.int32), e2],
                                   g2)
            plsc.addupdate_scatter(part_v, [jnp.ones((_L,), jnp.int32), e1],
                                   ones)
            plsc.addupdate_scatter(part_v, [jnp.ones((_L,), jnp.int32), e2],
                                   jnp.where(g2 > 0.0, ones, zeros))
            return 0

        lax.fori_loop(0, n_groups, group_body, 0)
        pltpu.sync_copy(out_v, gates_hbm.at[pl.ds(base, tpw), :])
        pltpu.sync_copy(part_v.at[pl.ds(0, 1), :],
                        parts_hbm.at[pl.ds(wid, 1), :])
        pltpu.sync_copy(part_v.at[pl.ds(1, 1), :],
                        parts_hbm.at[pl.ds(_NW + wid, 1), :])

    return route


def _combine_body(p_ref, aux_ref):
    n_blocks = p_ref.shape[0] // (2 * _NW)
    imp = jnp.zeros((_NUM_EXPERTS,), jnp.float32)
    ld = jnp.zeros((_NUM_EXPERTS,), jnp.float32)
    for b in range(n_blocks):
        o = b * 2 * _NW
        imp = imp + jnp.sum(p_ref[o:o + _NW, :], axis=0)
        ld = ld + jnp.sum(p_ref[o + _NW:o + 2 * _NW, :], axis=0)
    ne = float(_NUM_EXPERTS)
    imp_mean = jnp.sum(imp) / ne
    ld_mean = jnp.sum(ld) / ne
    imp_var = jnp.sum((imp - imp_mean) ** 2) / (ne - 1.0)
    ld_var = jnp.sum((ld - ld_mean) ** 2) / (ne - 1.0)
    aux_ref[0, 0] = _COEF * (imp_var / (imp_mean * imp_mean + _EPS)
                             + ld_var / (ld_mean * ld_mean + _EPS))


def _tc_combine(parts):
    return pl.pallas_call(
        _combine_body,
        out_specs=pl.BlockSpec(memory_space=pltpu.SMEM),
        out_shape=jax.ShapeDtypeStruct((1, 1), jnp.float32),
    )(parts)


def kernel(x, w_gate):
    n_tokens, _ = x.shape
    n_chunks = 2
    ct = n_tokens // n_chunks
    route = _make_sc_route(ct)
    gates_chunks = []
    parts_chunks = []
    for c in range(n_chunks):
        logits_c = _tc_logits(lax.slice_in_dim(x, c * ct, (c + 1) * ct), w_gate)
        g_c, p_c = route(logits_c)
        gates_chunks.append(g_c)
        parts_chunks.append(p_c)
    gates = jnp.concatenate(gates_chunks, axis=0)
    parts = jnp.concatenate(parts_chunks, axis=0)
    aux = _tc_combine(parts)
    return gates, aux.reshape(())


# speedup vs baseline: 3.1344x; 3.1344x over previous
import jax
import jax.numpy as jnp
from jax import lax
from jax.experimental import pallas as pl
from jax.experimental.pallas import tpu as pltpu

_NUM_EXPERTS = 16
_K = 2
_COEF = 0.01
_EPS = 1e-10


def _gating_body(xl_ref, xr_ref, w_ref, gates_ref, aux_ref, imp_ref, load_ref):
    i = pl.program_id(0)
    n = pl.num_programs(0)

    @pl.when(i == 0)
    def _init():
        imp_ref[...] = jnp.zeros_like(imp_ref)
        load_ref[...] = jnp.zeros_like(load_ref)

    h = xl_ref.shape[1]
    logits = (jnp.dot(xl_ref[...], w_ref[0:h, :],
                      preferred_element_type=jnp.float32)
              + jnp.dot(xr_ref[...], w_ref[h:2 * h, :],
                        preferred_element_type=jnp.float32))

    cols = logits.shape[1]
    tri_r = lax.broadcasted_iota(jnp.int32, (cols, cols), 0)
    tri_c = lax.broadcasted_iota(jnp.int32, (cols, cols), 1)
    tri = (tri_r <= tri_c).astype(jnp.float32)

    m1 = jnp.max(logits, axis=1, keepdims=True)
    eq1 = (logits == m1).astype(jnp.float32)
    c1 = jnp.dot(eq1, tri, preferred_element_type=jnp.float32)
    first1 = (eq1 * c1) == 1.0
    masked = jnp.where(first1, -jnp.inf, logits)
    m2 = jnp.max(masked, axis=1, keepdims=True)
    eq2 = (masked == m2).astype(jnp.float32)
    c2 = jnp.dot(eq2, tri, preferred_element_type=jnp.float32)
    first2 = (eq2 * c2) == 1.0

    d = jnp.exp(m2 - m1)
    s = 1.0 + d
    g1 = 1.0 / s
    g2 = d / s

    gates = jnp.where(first1, g1, jnp.where(first2, g2, 0.0))
    gates_ref[...] = gates

    imp_ref[...] += jnp.sum(gates, axis=0, keepdims=True)
    load_ref[...] += jnp.sum((gates > 0.0).astype(jnp.float32), axis=0,
                             keepdims=True)

    @pl.when(i == n - 1)
    def _finish():
        ne = float(_NUM_EXPERTS)
        imp = imp_ref[0, :]
        ld = load_ref[0, :]
        imp_mean = jnp.sum(imp) / ne
        ld_mean = jnp.sum(ld) / ne
        imp_var = jnp.sum((imp - imp_mean) ** 2) / (ne - 1.0)
        ld_var = jnp.sum((ld - ld_mean) ** 2) / (ne - 1.0)
        aux = _COEF * (imp_var / (imp_mean * imp_mean + _EPS)
                       + ld_var / (ld_mean * ld_mean + _EPS))
        aux_ref[0, 0] = aux


def kernel(x, w_gate):
    n_tokens, d_model = x.shape
    tile = 2048
    grid = n_tokens // tile
    h = d_model // 2

    gates, aux = pl.pallas_call(
        _gating_body,
        grid=(grid,),
        in_specs=[
            pl.BlockSpec((tile, h), lambda i: (i, 0)),
            pl.BlockSpec((tile, h), lambda i: (i, 1)),
            pl.BlockSpec((d_model, _NUM_EXPERTS), lambda i: (0, 0)),
        ],
        out_specs=[
            pl.BlockSpec((tile, _NUM_EXPERTS), lambda i: (i, 0)),
            pl.BlockSpec(memory_space=pltpu.SMEM),
        ],
        out_shape=[
            jax.ShapeDtypeStruct((n_tokens, _NUM_EXPERTS), jnp.float32),
            jax.ShapeDtypeStruct((1, 1), jnp.float32),
        ],
        scratch_shapes=[
            pltpu.VMEM((1, _NUM_EXPERTS), jnp.float32),
            pltpu.VMEM((1, _NUM_EXPERTS), jnp.float32),
        ],
    )(x, x, w_gate)
    return gates, aux.reshape(())


# speedup vs baseline: 3.1360x; 1.0005x over previous
import jax
import jax.numpy as jnp
from jax import lax
from jax.experimental import pallas as pl
from jax.experimental.pallas import tpu as pltpu

_NUM_EXPERTS = 16
_K = 2
_COEF = 0.01
_EPS = 1e-10


def _gating_body(x_ref, w_ref, gates_ref, aux_ref, imp_ref, load_ref):
    i = pl.program_id(0)
    n = pl.num_programs(0)

    @pl.when(i == 0)
    def _init():
        imp_ref[...] = jnp.zeros_like(imp_ref)
        load_ref[...] = jnp.zeros_like(load_ref)

    logits = jnp.dot(x_ref[...], w_ref[...],
                     preferred_element_type=jnp.float32)

    cols = logits.shape[1]
    tri_r = lax.broadcasted_iota(jnp.int32, (cols, cols), 0)
    tri_c = lax.broadcasted_iota(jnp.int32, (cols, cols), 1)
    tri = (tri_r <= tri_c).astype(jnp.float32)

    m1 = jnp.max(logits, axis=1, keepdims=True)
    eq1 = (logits == m1).astype(jnp.float32)
    c1 = jnp.dot(eq1, tri, preferred_element_type=jnp.float32)
    first1 = (eq1 * c1) == 1.0
    masked = jnp.where(first1, -jnp.inf, logits)
    m2 = jnp.max(masked, axis=1, keepdims=True)
    eq2 = (masked == m2).astype(jnp.float32)
    c2 = jnp.dot(eq2, tri, preferred_element_type=jnp.float32)
    first2 = (eq2 * c2) == 1.0

    d = jnp.exp(m2 - m1)
    s = 1.0 + d
    g1 = 1.0 / s
    g2 = d / s

    gates = jnp.where(first1, g1, jnp.where(first2, g2, 0.0))
    gates_ref[...] = gates

    imp_ref[...] += jnp.sum(gates, axis=0, keepdims=True)
    load_ref[...] += jnp.sum((gates > 0.0).astype(jnp.float32), axis=0,
                             keepdims=True)

    @pl.when(i == n - 1)
    def _finish():
        ne = float(_NUM_EXPERTS)
        imp = imp_ref[0, :]
        ld = load_ref[0, :]
        imp_mean = jnp.sum(imp) / ne
        ld_mean = jnp.sum(ld) / ne
        imp_var = jnp.sum((imp - imp_mean) ** 2) / (ne - 1.0)
        ld_var = jnp.sum((ld - ld_mean) ** 2) / (ne - 1.0)
        aux = _COEF * (imp_var / (imp_mean * imp_mean + _EPS)
                       + ld_var / (ld_mean * ld_mean + _EPS))
        aux_ref[0, 0] = aux


def kernel(x, w_gate):
    n_tokens, d_model = x.shape
    tile = 2048
    grid = n_tokens // tile

    gates, aux = pl.pallas_call(
        _gating_body,
        grid=(grid,),
        in_specs=[
            pl.BlockSpec((tile, d_model), lambda i: (i, 0)),
            pl.BlockSpec((d_model, _NUM_EXPERTS), lambda i: (0, 0)),
        ],
        out_specs=[
            pl.BlockSpec((tile, _NUM_EXPERTS), lambda i: (i, 0)),
            pl.BlockSpec(memory_space=pltpu.SMEM),
        ],
        out_shape=[
            jax.ShapeDtypeStruct((n_tokens, _NUM_EXPERTS), jnp.float32),
            jax.ShapeDtypeStruct((1, 1), jnp.float32),
        ],
        scratch_shapes=[
            pltpu.VMEM((1, _NUM_EXPERTS), jnp.float32),
            pltpu.VMEM((1, _NUM_EXPERTS), jnp.float32),
        ],
    )(x, w_gate)
    return gates, aux.reshape(())
